# baseline (speedup 1.0000x reference)
import functools

import jax
import jax.numpy as jnp
from jax import lax
from jax.experimental import pallas as pl
from jax.experimental.pallas import tpu as pltpu
from jax.experimental.pallas import tpu_sc as plsc

N, D = 8192, 1024
BM = 1024
TR, TCOL = 16, 512
NW = 32
CHUNK = N // NW
BATCH = 64
NB = CHUNK // BATCH


def _route_plan_body(x_ref, wdec_ref, bdec_ref, pos_ref, nl_ref):
    dec = jnp.dot(x_ref[...], wdec_ref[...],
                  preferred_element_type=jnp.float32)
    dec = dec + bdec_ref[0, 0]
    d = (dec > 0.0).astype(jnp.float32)
    d2 = d.reshape(TR, TCOL)
    dl2 = 1.0 - d2
    col = lax.broadcasted_iota(jnp.int32, (TCOL, TCOL), 0)
    row = lax.broadcasted_iota(jnp.int32, (TCOL, TCOL), 1)
    tri = (col <= row).astype(jnp.float32)
    sL = jnp.dot(dl2, tri, preferred_element_type=jnp.float32)
    sR = jnp.dot(d2, tri, preferred_element_type=jnp.float32)
    rowL = sL[:, TCOL - 1:TCOL]
    rowR = sR[:, TCOL - 1:TCOL]
    colt = lax.broadcasted_iota(jnp.int32, (TR, TR), 0)
    rowt = lax.broadcasted_iota(jnp.int32, (TR, TR), 1)
    stri = (rowt < colt).astype(jnp.float32)
    offL = jnp.dot(stri, rowL, preferred_element_type=jnp.float32,
                   precision=jax.lax.Precision.HIGHEST)
    offR = jnp.dot(stri, rowR, preferred_element_type=jnp.float32,
                   precision=jax.lax.Precision.HIGHEST)
    nl = offL[TR - 1, 0] + rowL[TR - 1, 0]
    cumLexcl = sL - dl2 + offL
    cumRexcl = sR - d2 + offR
    posf = jnp.where(d2 == 0.0, cumLexcl, nl + cumRexcl)
    pos_ref[...] = posf.astype(jnp.int32)
    nl_ref[...] = jnp.full((1, 128), nl, jnp.float32).astype(jnp.int32)


def _route_plan(x, wdec2, bdec2):
    return pl.pallas_call(
        _route_plan_body,
        in_specs=[
            pl.BlockSpec((N, D), lambda: (0, 0)),
            pl.BlockSpec((D, 1), lambda: (0, 0)),
            pl.BlockSpec((1, 1), lambda: (0, 0)),
        ],
        out_specs=[
            pl.BlockSpec((TR, TCOL), lambda: (0, 0)),
            pl.BlockSpec((1, 128), lambda: (0, 0)),
        ],
        out_shape=[
            jax.ShapeDtypeStruct((TR, TCOL), jnp.int32),
            jax.ShapeDtypeStruct((1, 128), jnp.int32),
        ],
    )(x, wdec2, bdec2)


def _sc_route(x, pos3):
    mesh = plsc.VectorSubcoreMesh(core_axis_name="c", subcore_axis_name="s")

    @functools.partial(
        pl.kernel,
        out_type=jax.ShapeDtypeStruct((N, D), jnp.float32),
        mesh=mesh,
        scratch_types=[
            pltpu.VMEM((NB, BATCH), jnp.int32),
            pltpu.VMEM((BATCH, D), jnp.float32),
            pltpu.SemaphoreType.DMA,
        ],
    )
    def k(x_hbm, pos_hbm, xp_hbm, posv, xbuf, sem):
        wid = lax.axis_index("s") * 2 + lax.axis_index("c")
        base = wid * CHUNK
        pltpu.sync_copy(pos_hbm.at[wid], posv)
        for b in range(NB):
            pltpu.sync_copy(x_hbm.at[pl.ds(base + b * BATCH, BATCH)], xbuf)
            pltpu.async_copy(xbuf, xp_hbm.at[posv.at[b]], sem).wait()

    return k(x, pos3)


def _moe_mm_body(nl_ref, xp_ref, wl_ref, bl_ref, wr_ref, br_ref, out_ref):
    nl = nl_ref[0]
    s = pl.program_id(0) * BM
    xb16 = xp_ref[...].astype(jnp.bfloat16)
    pure_l = s + BM <= nl
    pure_r = s >= nl

    @pl.when(pure_l)
    def _():
        out_ref[...] = (
            jnp.dot(xb16, wl_ref[...].astype(jnp.bfloat16),
                    preferred_element_type=jnp.float32) + bl_ref[...])

    @pl.when(pure_r)
    def _():
        out_ref[...] = (
            jnp.dot(xb16, wr_ref[...].astype(jnp.bfloat16),
                    preferred_element_type=jnp.float32) + br_ref[...])

    @pl.when(jnp.logical_not(jnp.logical_or(pure_l, pure_r)))
    def _():
        left = jnp.dot(xb16, wl_ref[...].astype(jnp.bfloat16),
                       preferred_element_type=jnp.float32) + bl_ref[...]
        right = jnp.dot(xb16, wr_ref[...].astype(jnp.bfloat16),
                        preferred_element_type=jnp.float32) + br_ref[...]
        ridx = lax.broadcasted_iota(jnp.int32, (BM, 1), 0) + s
        out_ref[...] = jnp.where(ridx < nl, left, right)


def _moe_mm(nl1, xp, W_left, bl2, W_right, br2):
    grid_spec = pltpu.PrefetchScalarGridSpec(
        num_scalar_prefetch=1,
        grid=(N // BM,),
        in_specs=[
            pl.BlockSpec((BM, D), lambda i, nl: (i, 0)),
            pl.BlockSpec((D, D), lambda i, nl: (0, 0)),
            pl.BlockSpec((1, D), lambda i, nl: (0, 0)),
            pl.BlockSpec((D, D), lambda i, nl: (0, 0)),
            pl.BlockSpec((1, D), lambda i, nl: (0, 0)),
        ],
        out_specs=pl.BlockSpec((BM, D), lambda i, nl: (i, 0)),
    )
    return pl.pallas_call(
        _moe_mm_body,
        grid_spec=grid_spec,
        out_shape=jax.ShapeDtypeStruct((N, D), jnp.float32),
    )(nl1, xp, W_left, bl2, W_right, br2)


def _sc_unroute(outp, pos3):
    mesh = plsc.VectorSubcoreMesh(core_axis_name="c", subcore_axis_name="s")

    @functools.partial(
        pl.kernel,
        out_type=jax.ShapeDtypeStruct((N, D), jnp.float32),
        mesh=mesh,
        scratch_types=[
            pltpu.VMEM((NB, BATCH), jnp.int32),
            pltpu.VMEM((BATCH, D), jnp.float32),
            pltpu.SemaphoreType.DMA,
        ],
    )
    def k(outp_hbm, pos_hbm, out_hbm, posv, buf, sem):
        wid = lax.axis_index("s") * 2 + lax.axis_index("c")
        base = wid * CHUNK
        pltpu.sync_copy(pos_hbm.at[wid], posv)
        for b in range(NB):
            pltpu.async_copy(outp_hbm.at[posv.at[b]], buf, sem).wait()
            pltpu.sync_copy(buf, out_hbm.at[pl.ds(base + b * BATCH, BATCH)])

    return k(outp, pos3)


def kernel(x, w_dec, b_dec, W_left, b_left, W_right, b_right):
    wdec2 = w_dec.reshape(D, 1)
    bdec2 = b_dec.reshape(1, 1)
    bl2 = b_left.reshape(1, D)
    br2 = b_right.reshape(1, D)
    pos, nlv = _route_plan(x, wdec2, bdec2)
    pos3 = pos.reshape(NW, NB, BATCH)
    nl1 = nlv[0, 0:1]
    xp = _sc_route(x, pos3)
    outp = _moe_mm(nl1, xp, W_left, bl2, W_right, br2)
    return _sc_unroute(outp, pos3)


# speedup vs baseline: 2.5198x; 2.5198x over previous
import jax
import jax.numpy as jnp
from jax.experimental import pallas as pl
from jax.experimental.pallas import tpu as pltpu

N, D = 8192, 1024
BM = 1024


def _fused_body(x_ref, wdec_ref, bdec_ref, wl_ref, bl_ref, wr_ref, br_ref,
                out_ref):
    xb = x_ref[...]
    dec = jnp.dot(xb, wdec_ref[...], preferred_element_type=jnp.float32)
    dec = dec + bdec_ref[0, 0]
    left = jnp.dot(xb, wl_ref[...], preferred_element_type=jnp.float32)
    left = left + bl_ref[...]
    right = jnp.dot(xb, wr_ref[...], preferred_element_type=jnp.float32)
    right = right + br_ref[...]
    out_ref[...] = jnp.where(dec > 0.0, right, left)


def kernel(x, w_dec, b_dec, W_left, b_left, W_right, b_right):
    wdec2 = w_dec.reshape(D, 1)
    bdec2 = b_dec.reshape(1, 1)
    bl2 = b_left.reshape(1, D)
    br2 = b_right.reshape(1, D)
    return pl.pallas_call(
        _fused_body,
        grid=(N // BM,),
        in_specs=[
            pl.BlockSpec((BM, D), lambda i: (i, 0)),
            pl.BlockSpec((D, 1), lambda i: (0, 0)),
            pl.BlockSpec((1, 1), lambda i: (0, 0)),
            pl.BlockSpec((D, D), lambda i: (0, 0)),
            pl.BlockSpec((1, D), lambda i: (0, 0)),
            pl.BlockSpec((D, D), lambda i: (0, 0)),
            pl.BlockSpec((1, D), lambda i: (0, 0)),
        ],
        out_specs=pl.BlockSpec((BM, D), lambda i: (i, 0)),
        out_shape=jax.ShapeDtypeStruct((N, D), jnp.float32),
    )(x, wdec2, bdec2, W_left, bl2, W_right, br2)


# speedup vs baseline: 2.5222x; 1.0010x over previous
import jax
import jax.numpy as jnp
from jax.experimental import pallas as pl

N, D = 8192, 1024
BM = 1024


def _fused_body(x_ref, wdec_ref, bdec_ref, wl_ref, bl_ref, wr_ref, br_ref,
                out_ref):
    xb = x_ref[...]
    dec = jnp.dot(xb, wdec_ref[...], preferred_element_type=jnp.float32)
    dec = dec + bdec_ref[0, 0]
    left = jnp.dot(xb, wl_ref[...], preferred_element_type=jnp.float32)
    left = left + bl_ref[...]
    right = jnp.dot(xb, wr_ref[...], preferred_element_type=jnp.float32)
    right = right + br_ref[...]
    out_ref[...] = jnp.where(dec > 0.0, right, left)


def kernel(x, w_dec, b_dec, W_left, b_left, W_right, b_right):
    wdec2 = w_dec.reshape(D, 1)
    bdec2 = b_dec.reshape(1, 1)
    bl2 = b_left.reshape(1, D)
    br2 = b_right.reshape(1, D)
    return pl.pallas_call(
        _fused_body,
        grid=(N // BM,),
        in_specs=[
            pl.BlockSpec((BM, D), lambda i: (i, 0)),
            pl.BlockSpec((D, 1), lambda i: (0, 0)),
            pl.BlockSpec((1, 1), lambda i: (0, 0)),
            pl.BlockSpec((D, D), lambda i: (0, 0)),
            pl.BlockSpec((1, D), lambda i: (0, 0)),
            pl.BlockSpec((D, D), lambda i: (0, 0)),
            pl.BlockSpec((1, D), lambda i: (0, 0)),
        ],
        out_specs=pl.BlockSpec((BM, D), lambda i: (i, 0)),
        out_shape=jax.ShapeDtypeStruct((N, D), jnp.float32),
    )(x, wdec2, bdec2, W_left, bl2, W_right, br2)
